# baseline (speedup 1.0000x reference)
import functools

import jax
import jax.numpy as jnp
from jax import lax
from jax.experimental import pallas as pl
from jax.experimental.pallas import tpu as pltpu
from jax.experimental.pallas import tpu_sc as plsc

_LANE_TILE = 128
_VREG = 16


def _table_body(bulk, x_ref, p_ref, w_ref, bias_ref, oa_ref, ob_ref):
    x = x_ref[...]
    p = p_ref[...]
    inv_d = 1.0 / x.shape[-1]
    for i in range(3):
        g = p[2 * i : 2 * i + 1, :]
        b = p[2 * i + 1 : 2 * i + 2, :]
        m = jnp.sum(x, axis=-1, keepdims=True) * inv_d
        ms = jnp.sum(x * x, axis=-1, keepdims=True) * inv_d
        x = (x - m) * lax.rsqrt(ms - m * m + 1e-5) * g + b
    y = jnp.dot(x, w_ref[...], preferred_element_type=jnp.float32) + bias_ref[...]
    n = y.shape[-1]
    oa_ref[...] = y[:, :bulk]
    ob_ref[...] = y[:, n - _LANE_TILE :]


@functools.cache
def _table_head(vocab, d, vocab_out):
    bulk = (vocab_out // _LANE_TILE) * _LANE_TILE
    return pl.pallas_call(
        functools.partial(_table_body, bulk),
        grid=(1,),
        in_specs=[
            pl.BlockSpec((vocab, d), lambda i: (0, 0)),
            pl.BlockSpec((6, d), lambda i: (0, 0)),
            pl.BlockSpec((d, vocab_out), lambda i: (0, 0)),
            pl.BlockSpec((1, vocab_out), lambda i: (0, 0)),
        ],
        out_specs=[
            pl.BlockSpec((vocab, bulk), lambda i: (0, 0)),
            pl.BlockSpec((vocab, _LANE_TILE), lambda i: (0, 0)),
        ],
        out_shape=[
            jax.ShapeDtypeStruct((vocab, bulk), jnp.float32),
            jax.ShapeDtypeStruct((vocab, _LANE_TILE), jnp.float32),
        ],
    )


@functools.cache
def _sc_expand(vocab, vocab_out, rows, cols):
    bulk = (vocab_out // _LANE_TILE) * _LANE_TILE
    toff = vocab_out - _LANE_TILE
    info = plsc.get_sparse_core_info()
    nw = info.num_cores * info.num_subcores
    assert rows % (2 * nw) == 0
    r_per_w = rows // nw

    mesh = plsc.VectorSubcoreMesh(core_axis_name="c", subcore_axis_name="s")

    @functools.partial(
        pl.kernel,
        mesh=mesh,
        out_type=jax.ShapeDtypeStruct((rows, cols, vocab_out), jnp.float32),
        scratch_types=[
            pltpu.VMEM((r_per_w, cols), jnp.int32),
            pltpu.VMEM((cols, bulk), jnp.float32),
            pltpu.VMEM((cols, _LANE_TILE), jnp.float32),
            pltpu.VMEM((cols, vocab_out), jnp.float32),
            pltpu.SemaphoreType.DMA,
            pltpu.SemaphoreType.DMA,
        ],
    )
    def expand(ta_hbm, tb_hbm, idx_hbm, out_hbm, idx_v, ba, bc, bb, sg, ss):
        wid = lax.axis_index("s") * info.num_cores + lax.axis_index("c")
        base = wid * r_per_w
        pltpu.sync_copy(idx_hbm.at[pl.ds(base, r_per_w), :], idx_v)

        def fire(j):
            pltpu.async_copy(ta_hbm.at[idx_v.at[j, :]], ba, sg)
            pltpu.async_copy(tb_hbm.at[idx_v.at[j, :]], bc, sg)

        def wait_gather():
            pltpu.make_async_copy(ta_hbm.at[idx_v.at[0, :]], ba, sg).wait()
            pltpu.make_async_copy(tb_hbm.at[idx_v.at[0, :]], bc, sg).wait()

        def fold():
            for r in range(cols):
                for k in range(bulk // _VREG):
                    bb[r, pl.ds(k * _VREG, _VREG)] = ba[r, pl.ds(k * _VREG, _VREG)]
                for k in range(_LANE_TILE // _VREG):
                    bb[r, pl.ds(toff + k * _VREG, _VREG)] = bc[
                        r, pl.ds(k * _VREG, _VREG)
                    ]

        fire(0)

        def body(j, _):
            wait_gather()

            @pl.when(j > 0)
            def _drain():
                pltpu.make_async_copy(bb, out_hbm.at[base + j - 1], ss).wait()

            fold()

            @pl.when(j + 1 < r_per_w)
            def _next():
                fire(j + 1)

            pltpu.async_copy(bb, out_hbm.at[base + j], ss)
            return _

        lax.fori_loop(0, r_per_w, body, 0)
        pltpu.make_async_copy(bb, out_hbm.at[base + r_per_w - 1], ss).wait()

    return expand


def kernel(indices, emb, g0, b0, g1, b1, gf, bf, W, bias):
    vocab, d = emb.shape
    vocab_out = W.shape[1]
    rows, cols = indices.shape
    params = jnp.stack([g0, b0, g1, b1, gf, bf], axis=0)

    ta, tb = _table_head(vocab, d, vocab_out)(
        emb, params, W, bias.reshape(1, vocab_out)
    )
    out = _sc_expand(vocab, vocab_out, rows, cols)(ta, tb, indices.astype(jnp.int32))
    return out


# speedup vs baseline: 1.8101x; 1.8101x over previous
import functools

import jax
import jax.numpy as jnp
from jax import lax
from jax.experimental import pallas as pl
from jax.experimental.pallas import tpu as pltpu
from jax.experimental.pallas import tpu_sc as plsc


@functools.cache
def _sc_gather(vocab, dpad, rows, cols, chunks):
    info = plsc.get_sparse_core_info()
    nw = info.num_cores * info.num_subcores
    assert rows % nw == 0 and dpad % 128 == 0
    r_per_w = rows // nw
    assert r_per_w % chunks == 0
    r_chunk = r_per_w // chunks

    mesh = plsc.VectorSubcoreMesh(core_axis_name="c", subcore_axis_name="s")

    @functools.partial(
        pl.kernel,
        mesh=mesh,
        out_type=jax.ShapeDtypeStruct((rows, cols, dpad), jnp.float32),
        scratch_types=[
            pltpu.VMEM((r_per_w, cols), jnp.int32),
            pltpu.VMEM((r_chunk, cols, dpad), jnp.float32),
            pltpu.SemaphoreType.DMA,
        ],
    )
    def gather(table_hbm, idx_hbm, out_hbm, idx_v, rows_v, sem):
        wid = lax.axis_index("s") * info.num_cores + lax.axis_index("c")
        base = wid * r_per_w
        pltpu.sync_copy(idx_hbm.at[pl.ds(base, r_per_w), :], idx_v)
        for c in range(chunks):
            copies = [
                pltpu.async_copy(
                    table_hbm.at[idx_v.at[c * r_chunk + j, :]], rows_v.at[j], sem
                )
                for j in range(r_chunk)
            ]
            for cp in copies:
                cp.wait()
            pltpu.sync_copy(rows_v, out_hbm.at[pl.ds(base + c * r_chunk, r_chunk)])

    return gather


def _head_body(d_real, x_ref, p_ref, w_ref, bias_ref, o_ref):
    x = x_ref[...]
    p = p_ref[...]
    inv_d = 1.0 / d_real
    for i in range(3):
        g = p[2 * i : 2 * i + 1, :]
        b = p[2 * i + 1 : 2 * i + 2, :]
        m = jnp.sum(x, axis=-1, keepdims=True) * inv_d
        ms = jnp.sum(x * x, axis=-1, keepdims=True) * inv_d
        x = (x - m) * lax.rsqrt(ms - m * m + 1e-5) * g + b
    w = w_ref[...]
    b = bias_ref[...]
    for r in range(o_ref.shape[0]):
        o_ref[r] = jnp.dot(x[r], w, preferred_element_type=jnp.float32) + b


@functools.cache
def _head(rows, cols, dpad, d, vocab_out, block_rows):
    grid = rows // block_rows
    return pl.pallas_call(
        functools.partial(_head_body, d),
        grid=(grid,),
        in_specs=[
            pl.BlockSpec((block_rows, cols, dpad), lambda i: (i, 0, 0)),
            pl.BlockSpec((6, dpad), lambda i: (0, 0)),
            pl.BlockSpec((dpad, vocab_out), lambda i: (0, 0)),
            pl.BlockSpec((1, vocab_out), lambda i: (0, 0)),
        ],
        out_specs=pl.BlockSpec((block_rows, cols, vocab_out), lambda i: (i, 0, 0)),
        out_shape=jax.ShapeDtypeStruct((rows, cols, vocab_out), jnp.float32),
    )


def kernel(indices, emb, g0, b0, g1, b1, gf, bf, W, bias):
    vocab, d = emb.shape
    vocab_out = W.shape[1]
    rows, cols = indices.shape
    dpad = 128
    emb_pad = jnp.pad(emb, ((0, 0), (0, dpad - d)))

    gathered = _sc_gather(vocab, dpad, rows, cols, 2)(emb_pad, indices.astype(jnp.int32))
    params = jnp.pad(jnp.stack([g0, b0, g1, b1, gf, bf], axis=0), ((0, 0), (0, dpad - d)))
    w_pad = jnp.pad(W, ((0, dpad - d), (0, 0)))
    out = _head(rows, cols, dpad, d, vocab_out, 64)(
        gathered, params, w_pad, bias.reshape(1, vocab_out)
    )
    return out
